# baseline (speedup 1.0000x reference)
import functools

import jax
import jax.numpy as jnp
from jax import lax
from jax.experimental import pallas as pl
from jax.experimental.pallas import tpu as pltpu
from jax.experimental.pallas import tpu_sc as plsc

N = 10000
E = 320000
F_IN = 128
H = 128
C = 40
CP = 48

NC, NS = 2, 16
NW = NC * NS
RPT = N // NS

B1 = 80
ITERS1 = 125
EPW1 = ITERS1 * B1

B2 = 128
ITERS2 = 79
EPW2 = ITERS2 * B2
EPAD2 = NW * EPW2
NHL = N + 8


def _sc_mesh():
    return plsc.VectorSubcoreMesh(core_axis_name="c", subcore_axis_name="s",
                                  num_cores=NC, num_subcores=NS)


_SC_PARAMS = pltpu.CompilerParams(use_tc_tiling_on_sc=False,
                                  needs_layout_passes=False)


@functools.cache
def _seg_sum_l1():
    @functools.partial(
        pl.kernel,
        mesh=_sc_mesh(),
        out_type=[
            jax.ShapeDtypeStruct((NC, N, H), jnp.float32),
            jax.ShapeDtypeStruct((NW, N), jnp.float32),
        ],
        scratch_types=[
            pltpu.VMEM((ITERS1, B1), jnp.int32),
            pltpu.VMEM((ITERS1, B1), jnp.int32),
            pltpu.VMEM((B1, H), jnp.float32),
            pltpu.VMEM((B1, H), jnp.float32),
            pltpu.VMEM((N,), jnp.float32),
            pltpu.VMEM_SHARED((N, H), jnp.float32),
            pltpu.SemaphoreType.DMA,
            pltpu.SemaphoreType.DMA,
        ],
        compiler_params=_SC_PARAMS,
    )
    def body_fn(x_hbm, src_hbm, dst_hbm, z_rows_hbm, z_hist_hbm,
                part_hbm, cntp_hbm,
                src2, dst2, rows0, rows1, hist, acc, sem0, sem1):
        c = lax.axis_index("c")
        s = lax.axis_index("s")
        wid = c * NS + s
        ones16 = jnp.ones((16,), jnp.float32)
        pltpu.sync_copy(src_hbm.at[pl.ds(wid * ITERS1, ITERS1)], src2)
        pltpu.sync_copy(dst_hbm.at[pl.ds(wid * ITERS1, ITERS1)], dst2)
        pltpu.sync_copy(z_rows_hbm, acc.at[pl.ds(s * RPT, RPT)])
        pltpu.sync_copy(z_hist_hbm, hist)
        plsc.subcore_barrier()

        def count(ii):
            for k in range(B1 // 16):
                idx = dst2[ii, pl.ds(k * 16, 16)]
                plsc.addupdate_scatter(hist, [idx], ones16)

        pltpu.async_copy(x_hbm.at[src2.at[0]], rows0, sem0)

        def body(j, carry):
            i0 = 2 * j
            i1 = i0 + 1
            pltpu.async_copy(x_hbm.at[src2.at[i1]], rows1, sem1)
            pltpu.make_async_copy(x_hbm.at[src2.at[i0]], rows0, sem0).wait()
            pltpu.sync_copy(rows0, acc.at[dst2.at[i0]], add=True)
            count(i0)
            pltpu.async_copy(x_hbm.at[src2.at[i0 + 2]], rows0, sem0)
            pltpu.make_async_copy(x_hbm.at[src2.at[i1]], rows1, sem1).wait()
            pltpu.sync_copy(rows1, acc.at[dst2.at[i1]], add=True)
            count(i1)
            return carry

        lax.fori_loop(0, ITERS1 // 2, body, 0)
        last = ITERS1 - 1
        pltpu.make_async_copy(x_hbm.at[src2.at[last]], rows0, sem0).wait()
        pltpu.sync_copy(rows0, acc.at[dst2.at[last]], add=True)
        count(last)
        plsc.subcore_barrier()
        pltpu.sync_copy(acc.at[pl.ds(s * RPT, RPT)],
                        part_hbm.at[c, pl.ds(s * RPT, RPT)])
        pltpu.sync_copy(hist, cntp_hbm.at[wid])

    return body_fn


@functools.cache
def _seg_sum_l2():
    @functools.partial(
        pl.kernel,
        mesh=_sc_mesh(),
        out_type=[jax.ShapeDtypeStruct((NC, N, CP), jnp.float32)],
        scratch_types=[
            pltpu.VMEM((ITERS2, B2), jnp.int32),
            pltpu.VMEM((ITERS2, B2), jnp.int32),
            pltpu.VMEM((B2, CP), jnp.float32),
            pltpu.VMEM((B2, CP), jnp.float32),
            pltpu.VMEM_SHARED((N, CP), jnp.float32),
            pltpu.SemaphoreType.DMA,
            pltpu.SemaphoreType.DMA,
        ],
        compiler_params=_SC_PARAMS,
    )
    def body_fn(hl_hbm, src_hbm, dst_hbm, z_rows_hbm,
                part_hbm,
                src2, dst2, rows0, rows1, acc, sem0, sem1):
        c = lax.axis_index("c")
        s = lax.axis_index("s")
        wid = c * NS + s
        pltpu.sync_copy(src_hbm.at[pl.ds(wid * ITERS2, ITERS2)], src2)
        pltpu.sync_copy(dst_hbm.at[pl.ds(wid * ITERS2, ITERS2)], dst2)
        pltpu.sync_copy(z_rows_hbm, acc.at[pl.ds(s * RPT, RPT)])
        plsc.subcore_barrier()

        pltpu.async_copy(hl_hbm.at[src2.at[0]], rows0, sem0)

        def body(j, carry):
            i0 = 2 * j
            i1 = i0 + 1
            pltpu.async_copy(hl_hbm.at[src2.at[i1]], rows1, sem1)
            pltpu.make_async_copy(hl_hbm.at[src2.at[i0]], rows0, sem0).wait()
            pltpu.sync_copy(rows0, acc.at[dst2.at[i0]], add=True)
            pltpu.async_copy(hl_hbm.at[src2.at[i0 + 2]], rows0, sem0)
            pltpu.make_async_copy(hl_hbm.at[src2.at[i1]], rows1, sem1).wait()
            pltpu.sync_copy(rows1, acc.at[dst2.at[i1]], add=True)
            return carry

        lax.fori_loop(0, ITERS2 // 2, body, 0)
        last = ITERS2 - 1
        pltpu.make_async_copy(hl_hbm.at[src2.at[last]], rows0, sem0).wait()
        pltpu.sync_copy(rows0, acc.at[dst2.at[last]], add=True)
        plsc.subcore_barrier()
        pltpu.sync_copy(acc.at[pl.ds(s * RPT, RPT)],
                        part_hbm.at[c, pl.ds(s * RPT, RPT)])

    return body_fn


BN = 1280

_DOT_T = (((1,), (1,)), ((), ()))


def _stage_b_body(p_ref, c_ref, x_ref, wl_ref, b_ref, wr_ref, w2_ref,
                  h_ref, hl_ref):
    tot = jnp.sum(c_ref[...], axis=0)
    den = jnp.maximum(tot, 1.0)[:, None]
    agg = (p_ref[0] + p_ref[1]) / den
    hb = lax.dot_general(agg, wl_ref[...], _DOT_T,
                         preferred_element_type=jnp.float32)
    hb += lax.dot_general(x_ref[...], wr_ref[...], _DOT_T,
                          preferred_element_type=jnp.float32)
    hb = jnp.maximum(hb + b_ref[...], 0.0)
    h_ref[...] = hb
    row = BN * pl.program_id(0) + lax.broadcasted_iota(jnp.int32, (BN, 1), 0)
    hl = lax.dot_general(hb, w2_ref[...], _DOT_T,
                         preferred_element_type=jnp.float32)
    hl_ref[...] = jnp.where(row < N, hl, 0.0)


_stage_b = pl.pallas_call(
    _stage_b_body,
    grid=(pl.cdiv(N, BN),),
    in_specs=[
        pl.BlockSpec((NC, BN, H), lambda i: (0, i, 0)),
        pl.BlockSpec((NW, BN), lambda i: (0, i)),
        pl.BlockSpec((BN, F_IN), lambda i: (i, 0)),
        pl.BlockSpec((H, F_IN), lambda i: (0, 0)),
        pl.BlockSpec((1, H), lambda i: (0, 0)),
        pl.BlockSpec((H, F_IN), lambda i: (0, 0)),
        pl.BlockSpec((CP, H), lambda i: (0, 0)),
    ],
    out_specs=[
        pl.BlockSpec((BN, H), lambda i: (i, 0)),
        pl.BlockSpec((BN, CP), lambda i: (i, 0)),
    ],
    out_shape=[
        jax.ShapeDtypeStruct((N, H), jnp.float32),
        jax.ShapeDtypeStruct((NHL, CP), jnp.float32),
    ],
)


def _stage_c_body(p_ref, c_ref, h_ref, b_ref, w_ref, o_ref):
    tot = jnp.sum(c_ref[...], axis=0)
    den = jnp.maximum(tot, 1.0)[:, None]
    agg = (p_ref[0] + p_ref[1])[:, :C] / den
    o = agg + b_ref[...] + lax.dot_general(h_ref[...], w_ref[...], _DOT_T,
                                           preferred_element_type=jnp.float32)
    m = jnp.max(o, axis=1, keepdims=True)
    sh = o - m
    lse = jnp.log(jnp.sum(jnp.exp(sh), axis=1, keepdims=True))
    o_ref[...] = sh - lse


_stage_c = pl.pallas_call(
    _stage_c_body,
    grid=(pl.cdiv(N, BN),),
    in_specs=[
        pl.BlockSpec((NC, BN, CP), lambda i: (0, i, 0)),
        pl.BlockSpec((NW, BN), lambda i: (0, i)),
        pl.BlockSpec((BN, H), lambda i: (i, 0)),
        pl.BlockSpec((1, C), lambda i: (0, 0)),
        pl.BlockSpec((C, H), lambda i: (0, 0)),
    ],
    out_specs=pl.BlockSpec((BN, C), lambda i: (i, 0)),
    out_shape=jax.ShapeDtypeStruct((N, C), jnp.float32),
)


def kernel(x, edge_index, W1l, b1l, W1r, W2l, b2l, W2r):
    x = x.astype(jnp.float32)
    e_src1 = edge_index[0].astype(jnp.int32).reshape(NW * ITERS1, B1)
    e_dst1 = edge_index[1].astype(jnp.int32).reshape(NW * ITERS1, B1)
    e_src2 = jnp.concatenate(
        [edge_index[0].astype(jnp.int32),
         jnp.full((EPAD2 - E,), N, jnp.int32)]).reshape(NW * ITERS2, B2)
    e_dst2 = jnp.concatenate(
        [edge_index[1].astype(jnp.int32),
         jnp.arange(EPAD2 - E, dtype=jnp.int32) % N]).reshape(NW * ITERS2, B2)

    z_rows1 = jnp.zeros((RPT, H), jnp.float32)
    z_hist = jnp.zeros((N,), jnp.float32)
    part1, cntp = _seg_sum_l1()(x, e_src1, e_dst1, z_rows1, z_hist)

    W2l_pad = jnp.zeros((CP, H), jnp.float32).at[:C].set(W2l)
    h, hl = _stage_b(part1, cntp, x, W1l, b1l.reshape(1, H), W1r, W2l_pad)

    z_rows2 = jnp.zeros((RPT, CP), jnp.float32)
    (part2,) = _seg_sum_l2()(hl, e_src2, e_dst2, z_rows2)

    return _stage_c(part2, cntp, h, b2l.reshape(1, C), W2r)


# speedup vs baseline: 1.1946x; 1.1946x over previous
import functools

import jax
import jax.numpy as jnp
from jax import lax
from jax.experimental import pallas as pl
from jax.experimental.pallas import tpu as pltpu
from jax.experimental.pallas import tpu_sc as plsc

N = 10000
E = 320000
F_IN = 128
H = 128
C = 40
CP = 48

NC, NS = 2, 16
NW = NC * NS
RPT = N // NS

B1 = 80
ITERS1 = 125
EPW1 = ITERS1 * B1

B2 = 80
ITERS2 = 125
EPW2 = ITERS2 * B2


def _sc_mesh():
    return plsc.VectorSubcoreMesh(core_axis_name="c", subcore_axis_name="s",
                                  num_cores=NC, num_subcores=NS)


_SC_PARAMS = pltpu.CompilerParams(use_tc_tiling_on_sc=False,
                                  needs_layout_passes=False)


@functools.cache
def _seg_sum_l1():
    @functools.partial(
        pl.kernel,
        mesh=_sc_mesh(),
        out_type=[
            jax.ShapeDtypeStruct((NC, N, H), jnp.float32),
            jax.ShapeDtypeStruct((NW, N), jnp.float32),
        ],
        scratch_types=[
            pltpu.VMEM((ITERS1, B1), jnp.int32),
            pltpu.VMEM((ITERS1, B1), jnp.int32),
            pltpu.VMEM((B1, H), jnp.float32),
            pltpu.VMEM((B1, H), jnp.float32),
            pltpu.VMEM((N,), jnp.float32),
            pltpu.VMEM_SHARED((N, H), jnp.float32),
            pltpu.SemaphoreType.DMA,
            pltpu.SemaphoreType.DMA,
        ],
        compiler_params=_SC_PARAMS,
    )
    def body_fn(x_hbm, src_hbm, dst_hbm, z_rows_hbm, z_hist_hbm,
                part_hbm, cntp_hbm,
                src2, dst2, rows0, rows1, hist, acc, sem0, sem1):
        c = lax.axis_index("c")
        s = lax.axis_index("s")
        wid = c * NS + s
        ones16 = jnp.ones((16,), jnp.float32)
        pltpu.sync_copy(src_hbm.at[pl.ds(wid * ITERS1, ITERS1)], src2)
        pltpu.sync_copy(dst_hbm.at[pl.ds(wid * ITERS1, ITERS1)], dst2)
        pltpu.sync_copy(z_rows_hbm, acc.at[pl.ds(s * RPT, RPT)])
        pltpu.sync_copy(z_hist_hbm, hist)
        plsc.subcore_barrier()

        def count(ii):
            for k in range(B1 // 16):
                idx = dst2[ii, pl.ds(k * 16, 16)]
                plsc.addupdate_scatter(hist, [idx], ones16)

        pltpu.async_copy(x_hbm.at[src2.at[0]], rows0, sem0)

        def body(j, carry):
            i0 = 2 * j
            i1 = i0 + 1
            pltpu.async_copy(x_hbm.at[src2.at[i1]], rows1, sem1)
            pltpu.make_async_copy(x_hbm.at[src2.at[i0]], rows0, sem0).wait()
            pltpu.sync_copy(rows0, acc.at[dst2.at[i0]], add=True)
            count(i0)
            pltpu.async_copy(x_hbm.at[src2.at[i0 + 2]], rows0, sem0)
            pltpu.make_async_copy(x_hbm.at[src2.at[i1]], rows1, sem1).wait()
            pltpu.sync_copy(rows1, acc.at[dst2.at[i1]], add=True)
            count(i1)
            return carry

        lax.fori_loop(0, ITERS1 // 2, body, 0)
        last = ITERS1 - 1
        pltpu.make_async_copy(x_hbm.at[src2.at[last]], rows0, sem0).wait()
        pltpu.sync_copy(rows0, acc.at[dst2.at[last]], add=True)
        count(last)
        plsc.subcore_barrier()
        pltpu.sync_copy(acc.at[pl.ds(s * RPT, RPT)],
                        part_hbm.at[c, pl.ds(s * RPT, RPT)])
        pltpu.sync_copy(hist, cntp_hbm.at[wid])

    return body_fn


@functools.cache
def _seg_sum_l2():
    @functools.partial(
        pl.kernel,
        mesh=_sc_mesh(),
        out_type=[jax.ShapeDtypeStruct((NC, N, CP), jnp.float32)],
        scratch_types=[
            pltpu.VMEM((ITERS2, B2), jnp.int32),
            pltpu.VMEM((ITERS2, B2), jnp.int32),
            pltpu.VMEM((B2, CP), jnp.float32),
            pltpu.VMEM((B2, CP), jnp.float32),
            pltpu.VMEM_SHARED((N, CP), jnp.float32),
            pltpu.SemaphoreType.DMA,
            pltpu.SemaphoreType.DMA,
        ],
        compiler_params=_SC_PARAMS,
    )
    def body_fn(hl_hbm, src_hbm, dst_hbm, z_rows_hbm,
                part_hbm,
                src2, dst2, rows0, rows1, acc, sem0, sem1):
        c = lax.axis_index("c")
        s = lax.axis_index("s")
        wid = c * NS + s
        pltpu.sync_copy(src_hbm.at[pl.ds(wid * ITERS2, ITERS2)], src2)
        pltpu.sync_copy(dst_hbm.at[pl.ds(wid * ITERS2, ITERS2)], dst2)
        pltpu.sync_copy(z_rows_hbm, acc.at[pl.ds(s * RPT, RPT)])
        plsc.subcore_barrier()

        pltpu.async_copy(hl_hbm.at[src2.at[0]], rows0, sem0)

        def body(j, carry):
            i0 = 2 * j
            i1 = i0 + 1
            pltpu.async_copy(hl_hbm.at[src2.at[i1]], rows1, sem1)
            pltpu.make_async_copy(hl_hbm.at[src2.at[i0]], rows0, sem0).wait()
            pltpu.sync_copy(rows0, acc.at[dst2.at[i0]], add=True)
            pltpu.async_copy(hl_hbm.at[src2.at[i0 + 2]], rows0, sem0)
            pltpu.make_async_copy(hl_hbm.at[src2.at[i1]], rows1, sem1).wait()
            pltpu.sync_copy(rows1, acc.at[dst2.at[i1]], add=True)
            return carry

        lax.fori_loop(0, ITERS2 // 2, body, 0)
        last = ITERS2 - 1
        pltpu.make_async_copy(hl_hbm.at[src2.at[last]], rows0, sem0).wait()
        pltpu.sync_copy(rows0, acc.at[dst2.at[last]], add=True)
        plsc.subcore_barrier()
        pltpu.sync_copy(acc.at[pl.ds(s * RPT, RPT)],
                        part_hbm.at[c, pl.ds(s * RPT, RPT)])

    return body_fn


BN = 1280

_DOT_T = (((1,), (1,)), ((), ()))


def _stage_b_body(p_ref, c_ref, x_ref, wl_ref, b_ref, wr_ref, w2_ref,
                  h_ref, hl_ref):
    tot = jnp.sum(c_ref[...], axis=0)
    den = jnp.maximum(tot, 1.0)[:, None]
    agg = (p_ref[0] + p_ref[1]) / den
    hb = lax.dot_general(agg, wl_ref[...], _DOT_T,
                         preferred_element_type=jnp.float32)
    hb += lax.dot_general(x_ref[...], wr_ref[...], _DOT_T,
                          preferred_element_type=jnp.float32)
    hb = jnp.maximum(hb + b_ref[...], 0.0)
    h_ref[...] = hb
    hl_ref[...] = lax.dot_general(hb, w2_ref[...], _DOT_T,
                                  preferred_element_type=jnp.float32)


_stage_b = pl.pallas_call(
    _stage_b_body,
    grid=(pl.cdiv(N, BN),),
    in_specs=[
        pl.BlockSpec((NC, BN, H), lambda i: (0, i, 0)),
        pl.BlockSpec((NW, BN), lambda i: (0, i)),
        pl.BlockSpec((BN, F_IN), lambda i: (i, 0)),
        pl.BlockSpec((H, F_IN), lambda i: (0, 0)),
        pl.BlockSpec((1, H), lambda i: (0, 0)),
        pl.BlockSpec((H, F_IN), lambda i: (0, 0)),
        pl.BlockSpec((CP, H), lambda i: (0, 0)),
    ],
    out_specs=[
        pl.BlockSpec((BN, H), lambda i: (i, 0)),
        pl.BlockSpec((BN, CP), lambda i: (i, 0)),
    ],
    out_shape=[
        jax.ShapeDtypeStruct((N, H), jnp.float32),
        jax.ShapeDtypeStruct((N, CP), jnp.float32),
    ],
)


def _stage_c_body(p_ref, c_ref, h_ref, b_ref, w_ref, o_ref):
    tot = jnp.sum(c_ref[...], axis=0)
    den = jnp.maximum(tot, 1.0)[:, None]
    agg = (p_ref[0] + p_ref[1])[:, :C] / den
    o = agg + b_ref[...] + lax.dot_general(h_ref[...], w_ref[...], _DOT_T,
                                           preferred_element_type=jnp.float32)
    m = jnp.max(o, axis=1, keepdims=True)
    sh = o - m
    lse = jnp.log(jnp.sum(jnp.exp(sh), axis=1, keepdims=True))
    o_ref[...] = sh - lse


_stage_c = pl.pallas_call(
    _stage_c_body,
    grid=(pl.cdiv(N, BN),),
    in_specs=[
        pl.BlockSpec((NC, BN, CP), lambda i: (0, i, 0)),
        pl.BlockSpec((NW, BN), lambda i: (0, i)),
        pl.BlockSpec((BN, H), lambda i: (i, 0)),
        pl.BlockSpec((1, C), lambda i: (0, 0)),
        pl.BlockSpec((C, H), lambda i: (0, 0)),
    ],
    out_specs=pl.BlockSpec((BN, C), lambda i: (i, 0)),
    out_shape=jax.ShapeDtypeStruct((N, C), jnp.float32),
)


def kernel(x, edge_index, W1l, b1l, W1r, W2l, b2l, W2r):
    x = x.astype(jnp.float32)
    e_src1 = edge_index[0].astype(jnp.int32).reshape(NW * ITERS1, B1)
    e_dst1 = edge_index[1].astype(jnp.int32).reshape(NW * ITERS1, B1)
    e_src2 = edge_index[0].astype(jnp.int32).reshape(NW * ITERS2, B2)
    e_dst2 = edge_index[1].astype(jnp.int32).reshape(NW * ITERS2, B2)

    z_rows1 = jnp.zeros((RPT, H), jnp.float32)
    z_hist = jnp.zeros((N,), jnp.float32)
    part1, cntp = _seg_sum_l1()(x, e_src1, e_dst1, z_rows1, z_hist)

    W2l_pad = jnp.zeros((CP, H), jnp.float32).at[:C].set(W2l)
    h, hl = _stage_b(part1, cntp, x, W1l, b1l.reshape(1, H), W1r, W2l_pad)

    z_rows2 = jnp.zeros((RPT, CP), jnp.float32)
    (part2,) = _seg_sum_l2()(hl, e_src2, e_dst2, z_rows2)

    return _stage_c(part2, cntp, h, b2l.reshape(1, C), W2r)


# speedup vs baseline: 1.2389x; 1.0371x over previous
import functools

import jax
import jax.numpy as jnp
from jax import lax
from jax.experimental import pallas as pl
from jax.experimental.pallas import tpu as pltpu
from jax.experimental.pallas import tpu_sc as plsc

N = 10000
E = 320000
F_IN = 128
H = 128
C = 40
CP = 48

NC, NS = 2, 16
NW = NC * NS
RPT = N // NS

B1 = 80
ITERS1 = 125
EPW1 = ITERS1 * B1

B2 = 80
ITERS2 = 125
EPW2 = ITERS2 * B2


def _sc_mesh():
    return plsc.VectorSubcoreMesh(core_axis_name="c", subcore_axis_name="s",
                                  num_cores=NC, num_subcores=NS)


_SC_PARAMS = pltpu.CompilerParams(use_tc_tiling_on_sc=False,
                                  needs_layout_passes=False)


@functools.cache
def _seg_sum_l1():
    @functools.partial(
        pl.kernel,
        mesh=_sc_mesh(),
        out_type=[
            jax.ShapeDtypeStruct((NC, N, H), jnp.float32),
            jax.ShapeDtypeStruct((NW, N), jnp.float32),
        ],
        scratch_types=[
            pltpu.VMEM((ITERS1, B1), jnp.int32),
            pltpu.VMEM((ITERS1, B1), jnp.int32),
            pltpu.VMEM((B1, H), jnp.float32),
            pltpu.VMEM((B1, H), jnp.float32),
            pltpu.VMEM((N,), jnp.float32),
            pltpu.VMEM_SHARED((N, H), jnp.float32),
            pltpu.SemaphoreType.DMA,
            pltpu.SemaphoreType.DMA,
        ],
        compiler_params=_SC_PARAMS,
    )
    def body_fn(x_hbm, src_hbm, dst_hbm, z_rows_hbm, z_hist_hbm,
                part_hbm, cntp_hbm,
                src2, dst2, rows0, rows1, hist, acc, sem0, sem1):
        c = lax.axis_index("c")
        s = lax.axis_index("s")
        wid = c * NS + s
        ones16 = jnp.ones((16,), jnp.float32)
        pltpu.sync_copy(src_hbm.at[pl.ds(wid * ITERS1, ITERS1)], src2)
        pltpu.sync_copy(dst_hbm.at[pl.ds(wid * ITERS1, ITERS1)], dst2)
        pltpu.sync_copy(z_rows_hbm, acc.at[pl.ds(s * RPT, RPT)])
        pltpu.sync_copy(z_hist_hbm, hist)
        plsc.subcore_barrier()

        def count(ii):
            for k in range(B1 // 16):
                idx = dst2[ii, pl.ds(k * 16, 16)]
                plsc.addupdate_scatter(hist, [idx], ones16)

        pltpu.async_copy(x_hbm.at[src2.at[0]], rows0, sem0)

        def body(j, carry):
            i0 = 2 * j
            i1 = i0 + 1
            pltpu.async_copy(x_hbm.at[src2.at[i1]], rows1, sem1)
            pltpu.make_async_copy(x_hbm.at[src2.at[i0]], rows0, sem0).wait()
            pltpu.sync_copy(rows0, acc.at[dst2.at[i0]], add=True)
            count(i0)
            pltpu.async_copy(x_hbm.at[src2.at[i0 + 2]], rows0, sem0)
            pltpu.make_async_copy(x_hbm.at[src2.at[i1]], rows1, sem1).wait()
            pltpu.sync_copy(rows1, acc.at[dst2.at[i1]], add=True)
            count(i1)
            return carry

        lax.fori_loop(0, ITERS1 // 2, body, 0)
        last = ITERS1 - 1
        pltpu.make_async_copy(x_hbm.at[src2.at[last]], rows0, sem0).wait()
        pltpu.sync_copy(rows0, acc.at[dst2.at[last]], add=True)
        count(last)
        plsc.subcore_barrier()
        pltpu.sync_copy(acc.at[pl.ds(s * RPT, RPT)],
                        part_hbm.at[c, pl.ds(s * RPT, RPT)])
        pltpu.sync_copy(hist, cntp_hbm.at[wid])

    return body_fn


@functools.cache
def _seg_sum_l2():
    @functools.partial(
        pl.kernel,
        mesh=_sc_mesh(),
        out_type=[jax.ShapeDtypeStruct((NC, N, CP), jnp.float32)],
        scratch_types=[
            pltpu.VMEM((ITERS2, B2), jnp.int32),
            pltpu.VMEM((ITERS2, B2), jnp.int32),
            pltpu.VMEM((B2, CP), jnp.float32),
            pltpu.VMEM((B2, CP), jnp.float32),
            pltpu.VMEM((B2, CP), jnp.float32),
            pltpu.VMEM((B2, CP), jnp.float32),
            pltpu.VMEM_SHARED((N, CP), jnp.float32),
            pltpu.SemaphoreType.DMA,
            pltpu.SemaphoreType.DMA,
            pltpu.SemaphoreType.DMA,
            pltpu.SemaphoreType.DMA,
            pltpu.SemaphoreType.DMA,
            pltpu.SemaphoreType.DMA,
            pltpu.SemaphoreType.DMA,
            pltpu.SemaphoreType.DMA,
        ],
        compiler_params=_SC_PARAMS,
    )
    def body_fn(hl_hbm, src_hbm, dst_hbm, z_rows_hbm,
                part_hbm,
                src2, dst2, r0, r1, r2, r3, acc,
                g0, g1, g2, g3, s0, s1, s2, s3):
        c = lax.axis_index("c")
        s = lax.axis_index("s")
        wid = c * NS + s
        rows = [r0, r1, r2, r3]
        gs = [g0, g1, g2, g3]
        ss = [s0, s1, s2, s3]
        pltpu.sync_copy(src_hbm.at[pl.ds(wid * ITERS2, ITERS2)], src2)
        pltpu.sync_copy(dst_hbm.at[pl.ds(wid * ITERS2, ITERS2)], dst2)
        pltpu.sync_copy(z_rows_hbm, acc.at[pl.ds(s * RPT, RPT)])
        plsc.subcore_barrier()

        def gather(cc, k):
            pltpu.async_copy(hl_hbm.at[src2.at[cc]], rows[k], gs[k])

        def gwait(cc, k):
            pltpu.make_async_copy(hl_hbm.at[src2.at[cc]], rows[k],
                                  gs[k]).wait()

        def scat(cc, k):
            pltpu.async_copy(rows[k], acc.at[dst2.at[cc]], ss[k], add=True)

        def swait(cc, k):
            pltpu.make_async_copy(rows[k], acc.at[dst2.at[cc]],
                                  ss[k]).wait()

        gather(0, 0)
        gather(1, 1)
        gwait(0, 0); scat(0, 0); gather(2, 2)
        gwait(1, 1); scat(1, 1); gather(3, 3)

        def body(j, carry):
            base = 4 * j + 2
            for u in range(4):
                cc = base + u
                k = (2 + u) % 4
                gwait(cc, k)
                scat(cc, k)
                swait(cc - 2, u)
                gather(cc + 2, u)
            return carry

        lax.fori_loop(0, (ITERS2 - 5) // 4, body, 0)
        gwait(122, 2); scat(122, 2); swait(120, 0); gather(124, 0)
        gwait(123, 3); scat(123, 3)
        gwait(124, 0); scat(124, 0)
        swait(121, 1)
        swait(122, 2)
        swait(123, 3)
        swait(124, 0)
        plsc.subcore_barrier()
        pltpu.sync_copy(acc.at[pl.ds(s * RPT, RPT)],
                        part_hbm.at[c, pl.ds(s * RPT, RPT)])

    return body_fn


BN = 1280

_DOT_T = (((1,), (1,)), ((), ()))


def _stage_b_body(p_ref, c_ref, x_ref, wl_ref, b_ref, wr_ref, w2_ref,
                  h_ref, hl_ref):
    tot = jnp.sum(c_ref[...], axis=0)
    den = jnp.maximum(tot, 1.0)[:, None]
    agg = (p_ref[0] + p_ref[1]) / den
    hb = lax.dot_general(agg, wl_ref[...], _DOT_T,
                         preferred_element_type=jnp.float32)
    hb += lax.dot_general(x_ref[...], wr_ref[...], _DOT_T,
                          preferred_element_type=jnp.float32)
    hb = jnp.maximum(hb + b_ref[...], 0.0)
    h_ref[...] = hb
    hl_ref[...] = lax.dot_general(hb, w2_ref[...], _DOT_T,
                                  preferred_element_type=jnp.float32)


_stage_b = pl.pallas_call(
    _stage_b_body,
    grid=(pl.cdiv(N, BN),),
    in_specs=[
        pl.BlockSpec((NC, BN, H), lambda i: (0, i, 0)),
        pl.BlockSpec((NW, BN), lambda i: (0, i)),
        pl.BlockSpec((BN, F_IN), lambda i: (i, 0)),
        pl.BlockSpec((H, F_IN), lambda i: (0, 0)),
        pl.BlockSpec((1, H), lambda i: (0, 0)),
        pl.BlockSpec((H, F_IN), lambda i: (0, 0)),
        pl.BlockSpec((CP, H), lambda i: (0, 0)),
    ],
    out_specs=[
        pl.BlockSpec((BN, H), lambda i: (i, 0)),
        pl.BlockSpec((BN, CP), lambda i: (i, 0)),
    ],
    out_shape=[
        jax.ShapeDtypeStruct((N, H), jnp.float32),
        jax.ShapeDtypeStruct((N, CP), jnp.float32),
    ],
)


def _stage_c_body(p_ref, c_ref, h_ref, b_ref, w_ref, o_ref):
    tot = jnp.sum(c_ref[...], axis=0)
    den = jnp.maximum(tot, 1.0)[:, None]
    agg = (p_ref[0] + p_ref[1])[:, :C] / den
    o = agg + b_ref[...] + lax.dot_general(h_ref[...], w_ref[...], _DOT_T,
                                           preferred_element_type=jnp.float32)
    m = jnp.max(o, axis=1, keepdims=True)
    sh = o - m
    lse = jnp.log(jnp.sum(jnp.exp(sh), axis=1, keepdims=True))
    o_ref[...] = sh - lse


_stage_c = pl.pallas_call(
    _stage_c_body,
    grid=(pl.cdiv(N, BN),),
    in_specs=[
        pl.BlockSpec((NC, BN, CP), lambda i: (0, i, 0)),
        pl.BlockSpec((NW, BN), lambda i: (0, i)),
        pl.BlockSpec((BN, H), lambda i: (i, 0)),
        pl.BlockSpec((1, C), lambda i: (0, 0)),
        pl.BlockSpec((C, H), lambda i: (0, 0)),
    ],
    out_specs=pl.BlockSpec((BN, C), lambda i: (i, 0)),
    out_shape=jax.ShapeDtypeStruct((N, C), jnp.float32),
)


def kernel(x, edge_index, W1l, b1l, W1r, W2l, b2l, W2r):
    x = x.astype(jnp.float32)
    e_src1 = edge_index[0].astype(jnp.int32).reshape(NW * ITERS1, B1)
    e_dst1 = edge_index[1].astype(jnp.int32).reshape(NW * ITERS1, B1)
    e_src2 = edge_index[0].astype(jnp.int32).reshape(NW * ITERS2, B2)
    e_dst2 = edge_index[1].astype(jnp.int32).reshape(NW * ITERS2, B2)

    z_rows1 = jnp.zeros((RPT, H), jnp.float32)
    z_hist = jnp.zeros((N,), jnp.float32)
    part1, cntp = _seg_sum_l1()(x, e_src1, e_dst1, z_rows1, z_hist)

    W2l_pad = jnp.zeros((CP, H), jnp.float32).at[:C].set(W2l)
    h, hl = _stage_b(part1, cntp, x, W1l, b1l.reshape(1, H), W1r, W2l_pad)

    z_rows2 = jnp.zeros((RPT, CP), jnp.float32)
    (part2,) = _seg_sum_l2()(hl, e_src2, e_dst2, z_rows2)

    return _stage_c(part2, cntp, h, b2l.reshape(1, C), W2r)
